# baseline (speedup 1.0000x reference)
import functools

import jax
import jax.numpy as jnp
from jax import lax
from jax.experimental import pallas as pl
from jax.experimental.pallas import tpu as pltpu
from jax.experimental.pallas import tpu_sc as plsc

EDGE_VOCAB = 64
EDGE_DIM = 128
HIDDEN_DIM = 256
B, N = 16, 64
B_TOT = B * N * N
N_REPLICAS = 32
TBL = EDGE_VOCAB * HIDDEN_DIM


def _mlp_table_kernel(table_ref, w1_ref, b1_ref, w2_ref, b2_ref, out_ref):
    h = jnp.dot(table_ref[...], w1_ref[...], preferred_element_type=jnp.float32)
    h = h + b1_ref[...]
    h = jax.nn.gelu(h)
    o = jnp.dot(h, w2_ref[...], preferred_element_type=jnp.float32)
    out_ref[...] = o + b2_ref[...]


def _fused_table(table, W1, b1, W2, b2):
    return pl.pallas_call(
        _mlp_table_kernel,
        grid=(N_REPLICAS,),
        in_specs=[
            pl.BlockSpec((EDGE_VOCAB, EDGE_DIM), lambda i: (0, 0)),
            pl.BlockSpec((EDGE_DIM, HIDDEN_DIM), lambda i: (0, 0)),
            pl.BlockSpec((1, HIDDEN_DIM), lambda i: (0, 0)),
            pl.BlockSpec((HIDDEN_DIM, HIDDEN_DIM), lambda i: (0, 0)),
            pl.BlockSpec((1, HIDDEN_DIM), lambda i: (0, 0)),
        ],
        out_specs=pl.BlockSpec((EDGE_VOCAB, HIDDEN_DIM), lambda i: (i, 0)),
        out_shape=jax.ShapeDtypeStruct(
            (N_REPLICAS * EDGE_VOCAB, HIDDEN_DIM), jnp.float32
        ),
    )(table, W1, b1.reshape(1, HIDDEN_DIM), W2, b2.reshape(1, HIDDEN_DIM))


def _make_gather():
    info = plsc.get_sparse_core_info()
    NC, NS, L = info.num_cores, info.num_subcores, info.num_lanes
    NW = NC * NS
    b_per_w = B_TOT // NW
    CHUNK = 128
    n_chunks = b_per_w // CHUNK
    groups = CHUNK // L
    mesh = plsc.VectorSubcoreMesh(core_axis_name="c", subcore_axis_name="s")

    @functools.partial(
        pl.kernel,
        mesh=mesh,
        compiler_params=pltpu.CompilerParams(needs_layout_passes=False),
        out_type=jax.ShapeDtypeStruct((B_TOT * HIDDEN_DIM,), jnp.float32),
        scratch_types=[
            pltpu.VMEM((b_per_w,), jnp.int32),
            pltpu.VMEM((TBL,), jnp.float32),
            pltpu.VMEM((CHUNK * HIDDEN_DIM,), jnp.float32),
            pltpu.VMEM((CHUNK * HIDDEN_DIM,), jnp.float32),
            pltpu.SemaphoreType.DMA,
            pltpu.SemaphoreType.DMA,
        ],
    )
    def gather_k(
        idx_hbm, table_hbm, out_hbm, idx_v, table_v, rows_a, rows_b, sem0, sem1
    ):
        wid = lax.axis_index("s") * NC + lax.axis_index("c")
        base = wid * b_per_w
        sems = (sem0, sem1)
        bufs = (rows_a, rows_b)
        pltpu.sync_copy(table_hbm.at[pl.ds(wid * TBL, TBL)], table_v)
        pltpu.sync_copy(idx_hbm.at[pl.ds(base, b_per_w)], idx_v)

        lane = lax.iota(jnp.int32, L)

        def fill_chunk(i, buf):
            def group_body(g, _):
                row_idx = idx_v[pl.ds(i * CHUNK + g * L, L)]
                src0 = row_idx * HIDDEN_DIM
                dst0 = (g * L + lane) * HIDDEN_DIM

                def f_body(f, _):
                    v = plsc.load_gather(table_v, [src0 + f])
                    plsc.store_scatter(buf, [dst0 + f], v)
                    return 0

                return plsc.parallel_loop(0, HIDDEN_DIM, 1, unroll=16, carry=_)(
                    f_body
                )

            lax.fori_loop(0, groups, group_body, 0)

        handles = {}
        for i in range(n_chunks):
            if i >= 2:
                handles.pop(i - 2).wait()
            fill_chunk(i, bufs[i % 2])
            handles[i] = pltpu.async_copy(
                bufs[i % 2],
                out_hbm.at[pl.ds((base + i * CHUNK) * HIDDEN_DIM, CHUNK * HIDDEN_DIM)],
                sems[i % 2],
            )
        for i in (n_chunks - 2, n_chunks - 1):
            handles.pop(i).wait()

    return gather_k


def kernel(edge_types, table, W1, b1, W2, b2):
    fused = _fused_table(table, W1, b1, W2, b2)
    idx = edge_types.reshape(B_TOT).astype(jnp.int32)
    out = _make_gather()(idx, fused.reshape(N_REPLICAS * TBL))
    return out.reshape(B, N, N, HIDDEN_DIM)


# speedup vs baseline: 3.8287x; 3.8287x over previous
import functools

import jax
import jax.numpy as jnp
from jax import lax
from jax.experimental import pallas as pl
from jax.experimental.pallas import tpu as pltpu
from jax.experimental.pallas import tpu_sc as plsc

EDGE_VOCAB = 64
EDGE_DIM = 128
HIDDEN_DIM = 256
B, N = 16, 64
B_TOT = B * N * N
N_REPLICAS = 32


def _mlp_table_kernel(table_ref, w1_ref, b1_ref, w2_ref, b2_ref, out_ref):
    h = jnp.dot(table_ref[...], w1_ref[...], preferred_element_type=jnp.float32)
    h = h + b1_ref[...]
    h = jax.nn.gelu(h)
    o = jnp.dot(h, w2_ref[...], preferred_element_type=jnp.float32)
    out_ref[...] = o + b2_ref[...]


def _fused_table(table, W1, b1, W2, b2):
    return pl.pallas_call(
        _mlp_table_kernel,
        grid=(N_REPLICAS,),
        in_specs=[
            pl.BlockSpec((EDGE_VOCAB, EDGE_DIM), lambda i: (0, 0)),
            pl.BlockSpec((EDGE_DIM, HIDDEN_DIM), lambda i: (0, 0)),
            pl.BlockSpec((1, HIDDEN_DIM), lambda i: (0, 0)),
            pl.BlockSpec((HIDDEN_DIM, HIDDEN_DIM), lambda i: (0, 0)),
            pl.BlockSpec((1, HIDDEN_DIM), lambda i: (0, 0)),
        ],
        out_specs=pl.BlockSpec((EDGE_VOCAB, HIDDEN_DIM), lambda i: (i, 0)),
        out_shape=jax.ShapeDtypeStruct(
            (N_REPLICAS * EDGE_VOCAB, HIDDEN_DIM), jnp.float32
        ),
    )(table, W1, b1.reshape(1, HIDDEN_DIM), W2, b2.reshape(1, HIDDEN_DIM))


N_BUF = 4
CHUNK = 64
GATHER_LAG = 2


def _make_gather():
    info = plsc.get_sparse_core_info()
    NC, NS = info.num_cores, info.num_subcores
    NW = NC * NS
    b_per_w = B_TOT // NW
    n_chunks = b_per_w // CHUNK
    mesh = plsc.VectorSubcoreMesh(core_axis_name="c", subcore_axis_name="s")

    @functools.partial(
        pl.kernel,
        mesh=mesh,
        out_type=jax.ShapeDtypeStruct((B_TOT, HIDDEN_DIM), jnp.float32),
        scratch_types=[
            pltpu.VMEM((b_per_w,), jnp.int32),
        ]
        + [pltpu.VMEM((CHUNK, HIDDEN_DIM), jnp.float32) for _ in range(N_BUF)]
        + [pltpu.SemaphoreType.DMA for _ in range(2 * N_BUF)],
    )
    def gather_k(idx_hbm, table_hbm, out_hbm, idx_v, *bufs_and_sems):
        bufs = bufs_and_sems[:N_BUF]
        g_sems = bufs_and_sems[N_BUF : 2 * N_BUF]
        w_sems = bufs_and_sems[2 * N_BUF :]
        wid = lax.axis_index("s") * NC + lax.axis_index("c")
        base = wid * b_per_w
        pltpu.sync_copy(idx_hbm.at[pl.ds(base, b_per_w)], idx_v)
        off = (wid * EDGE_VOCAB).astype(jnp.int32)
        for j in range(b_per_w // 16):
            sl = pl.ds(j * 16, 16)
            idx_v[sl] = idx_v[sl] + off

        def start_gather(i):
            return pltpu.async_copy(
                table_hbm.at[idx_v.at[pl.ds(i * CHUNK, CHUNK)]],
                bufs[i % N_BUF],
                g_sems[i % N_BUF],
            )

        def start_write(i):
            return pltpu.async_copy(
                bufs[i % N_BUF],
                out_hbm.at[pl.ds(base + i * CHUNK, CHUNK)],
                w_sems[i % N_BUF],
            )

        g_handles, w_handles = {}, {}
        for s in range(n_chunks + GATHER_LAG):
            i_g = s
            if i_g < n_chunks:
                if i_g >= N_BUF:
                    w_handles.pop(i_g - N_BUF).wait()
                g_handles[i_g] = start_gather(i_g)
            i_w = s - GATHER_LAG
            if 0 <= i_w < n_chunks:
                g_handles.pop(i_w).wait()
                w_handles[i_w] = start_write(i_w)
        for i in sorted(w_handles):
            w_handles.pop(i).wait()

    return gather_k


def kernel(edge_types, table, W1, b1, W2, b2):
    fused = _fused_table(table, W1, b1, W2, b2)
    idx = edge_types.reshape(B_TOT).astype(jnp.int32)
    out = _make_gather()(idx, fused)
    return out.reshape(B, N, N, HIDDEN_DIM)


# speedup vs baseline: 4.4610x; 1.1651x over previous
import functools

import jax
import jax.numpy as jnp
from jax import lax
from jax.experimental import pallas as pl
from jax.experimental.pallas import tpu as pltpu
from jax.experimental.pallas import tpu_sc as plsc

EDGE_VOCAB = 64
EDGE_DIM = 128
HIDDEN_DIM = 256
B, N = 16, 64
B_TOT = B * N * N
N_REPLICAS = 32


def _mlp_table_kernel(table_ref, w1_ref, b1_ref, w2_ref, b2_ref, out_ref):
    h = jnp.dot(table_ref[...], w1_ref[...], preferred_element_type=jnp.float32)
    h = h + b1_ref[...]
    h = jax.nn.gelu(h)
    o = jnp.dot(h, w2_ref[...], preferred_element_type=jnp.float32)
    o = o + b2_ref[...]
    out_ref[...] = jnp.broadcast_to(
        o[None], (N_REPLICAS, EDGE_VOCAB, HIDDEN_DIM)
    ).reshape(N_REPLICAS * EDGE_VOCAB, HIDDEN_DIM)


def _fused_table(table, W1, b1, W2, b2):
    return pl.pallas_call(
        _mlp_table_kernel,
        out_shape=jax.ShapeDtypeStruct(
            (N_REPLICAS * EDGE_VOCAB, HIDDEN_DIM), jnp.float32
        ),
    )(table, W1, b1.reshape(1, HIDDEN_DIM), W2, b2.reshape(1, HIDDEN_DIM))


N_BUF = 4
CHUNK = 64
GATHER_LAG = 2


def _make_gather():
    info = plsc.get_sparse_core_info()
    NC, NS = info.num_cores, info.num_subcores
    NW = NC * NS
    b_per_w = B_TOT // NW
    n_chunks = b_per_w // CHUNK
    mesh = plsc.VectorSubcoreMesh(core_axis_name="c", subcore_axis_name="s")

    @functools.partial(
        pl.kernel,
        mesh=mesh,
        out_type=jax.ShapeDtypeStruct((B_TOT, HIDDEN_DIM), jnp.float32),
        scratch_types=[
            pltpu.VMEM((b_per_w,), jnp.int32),
        ]
        + [pltpu.VMEM((CHUNK, HIDDEN_DIM), jnp.float32) for _ in range(N_BUF)]
        + [pltpu.SemaphoreType.DMA for _ in range(2 * N_BUF)],
    )
    def gather_k(idx_hbm, table_hbm, out_hbm, idx_v, *bufs_and_sems):
        bufs = bufs_and_sems[:N_BUF]
        g_sems = bufs_and_sems[N_BUF : 2 * N_BUF]
        w_sems = bufs_and_sems[2 * N_BUF :]
        wid = lax.axis_index("s") * NC + lax.axis_index("c")
        base = wid * b_per_w
        pltpu.sync_copy(idx_hbm.at[pl.ds(base, b_per_w)], idx_v)
        off = (wid * EDGE_VOCAB).astype(jnp.int32)
        for j in range(b_per_w // 16):
            sl = pl.ds(j * 16, 16)
            idx_v[sl] = idx_v[sl] + off

        def start_gather(i):
            return pltpu.async_copy(
                table_hbm.at[idx_v.at[pl.ds(i * CHUNK, CHUNK)]],
                bufs[i % N_BUF],
                g_sems[i % N_BUF],
            )

        def start_write(i):
            return pltpu.async_copy(
                bufs[i % N_BUF],
                out_hbm.at[pl.ds(base + i * CHUNK, CHUNK)],
                w_sems[i % N_BUF],
            )

        g_handles, w_handles = {}, {}
        for s in range(n_chunks + GATHER_LAG):
            i_g = s
            if i_g < n_chunks:
                if i_g >= N_BUF:
                    w_handles.pop(i_g - N_BUF).wait()
                g_handles[i_g] = start_gather(i_g)
            i_w = s - GATHER_LAG
            if 0 <= i_w < n_chunks:
                g_handles.pop(i_w).wait()
                w_handles[i_w] = start_write(i_w)
        for i in sorted(w_handles):
            w_handles.pop(i).wait()

    return gather_k


def kernel(edge_types, table, W1, b1, W2, b2):
    fused = _fused_table(table, W1, b1, W2, b2)
    idx = edge_types.reshape(B_TOT).astype(jnp.int32)
    out = _make_gather()(idx, fused)
    return out.reshape(B, N, N, HIDDEN_DIM)
